# baseline (speedup 1.0000x reference)
import jax
import jax.numpy as jnp
from jax.experimental import pallas as pl
from jax.experimental.pallas import tpu as pltpu

_DIM = 128
_NH = 8
_PACK = 16
_BM = 256


def _lsh_block(xp_ref, rvb_ref, o_ref):
    proj = jnp.dot(xp_ref[...], rvb_ref[...], preferred_element_type=jnp.float32)
    buckets = jnp.floor(proj).astype(jnp.int32) & 1023
    o_ref[...] = buckets.astype(jnp.float32)


def kernel(x, random_vectors):
    n = x.shape[0]
    npacked = n // _PACK
    xp = x.reshape(npacked, _PACK * _DIM)
    eye = jnp.eye(_PACK, dtype=jnp.float32)
    rv_big = jnp.einsum("jk,dh->jdkh", eye, random_vectors).reshape(
        _PACK * _DIM, _PACK * _NH
    )
    grid = (pl.cdiv(npacked, _BM),)
    packed = pl.pallas_call(
        _lsh_block,
        grid=grid,
        in_specs=[
            pl.BlockSpec((_BM, _PACK * _DIM), lambda i: (i, 0)),
            pl.BlockSpec((_PACK * _DIM, _PACK * _NH), lambda i: (0, 0)),
        ],
        out_specs=pl.BlockSpec((_BM, _PACK * _NH), lambda i: (i, 0)),
        out_shape=jax.ShapeDtypeStruct((npacked, _PACK * _NH), jnp.float32),
        compiler_params=pltpu.CompilerParams(dimension_semantics=("parallel",)),
    )(xp, rv_big)
    return packed.reshape(n, _NH)


# speedup vs baseline: 2.2068x; 2.2068x over previous
import jax
import jax.numpy as jnp
from jax.experimental import pallas as pl
from jax.experimental.pallas import tpu as pltpu

_DIM = 128
_NH = 8
_BLK = 12800


def _lsh_block(x_ref, rv_ref, o_ref):
    proj_t = jax.lax.dot_general(
        rv_ref[...], x_ref[...],
        dimension_numbers=(((0,), (1,)), ((), ())),
        preferred_element_type=jnp.float32,
    )
    buckets = jnp.floor(proj_t).astype(jnp.int32) & 1023
    o_ref[...] = buckets.astype(jnp.float32).T


def kernel(x, random_vectors):
    n = x.shape[0]
    grid = (pl.cdiv(n, _BLK),)
    return pl.pallas_call(
        _lsh_block,
        grid=grid,
        in_specs=[
            pl.BlockSpec((_BLK, _DIM), lambda i: (i, 0)),
            pl.BlockSpec((_DIM, _NH), lambda i: (0, 0)),
        ],
        out_specs=pl.BlockSpec((_BLK, _NH), lambda i: (i, 0)),
        out_shape=jax.ShapeDtypeStruct((n, _NH), jnp.float32),
        compiler_params=pltpu.CompilerParams(dimension_semantics=("parallel",)),
    )(x, random_vectors)


# speedup vs baseline: 2.2113x; 1.0021x over previous
import jax
import jax.numpy as jnp
from jax.experimental import pallas as pl
from jax.experimental.pallas import tpu as pltpu

_DIM = 128
_NH = 8
_NSPLIT = 4
_SUB = 5000
_BLK = _NSPLIT * _SUB


def _lsh_block(x0_ref, x1_ref, x2_ref, x3_ref, rv_ref, o_ref):
    rv = rv_ref[...]
    for k, xr in enumerate((x0_ref, x1_ref, x2_ref, x3_ref)):
        proj_t = jax.lax.dot_general(
            rv, xr[...],
            dimension_numbers=(((0,), (1,)), ((), ())),
            preferred_element_type=jnp.float32,
        )
        buckets = jnp.floor(proj_t).astype(jnp.int32) & 1023
        o_ref[k * _SUB:(k + 1) * _SUB, :] = buckets.astype(jnp.float32).T


def kernel(x, random_vectors):
    n = x.shape[0]
    grid = (pl.cdiv(n, _BLK),)
    in_specs = [
        pl.BlockSpec((_SUB, _DIM), lambda i, k=k: (_NSPLIT * i + k, 0))
        for k in range(_NSPLIT)
    ]
    in_specs.append(pl.BlockSpec((_DIM, _NH), lambda i: (0, 0)))
    return pl.pallas_call(
        _lsh_block,
        grid=grid,
        in_specs=in_specs,
        out_specs=pl.BlockSpec((_BLK, _NH), lambda i: (i, 0)),
        out_shape=jax.ShapeDtypeStruct((n, _NH), jnp.float32),
        compiler_params=pltpu.CompilerParams(dimension_semantics=("parallel",)),
    )(x, x, x, x, random_vectors)
